# baseline (speedup 1.0000x reference)
import functools

import jax
import jax.numpy as jnp
from jax import lax
from jax.experimental import pallas as pl
from jax.experimental.pallas import tpu as pltpu
from jax.experimental.pallas import tpu_sc as plsc

N = 10000
E = 320000
FEAT = 128
H = 8
C = 16
OUT = 64

NC = 2
NS = 16
NW = NC * NS
NPAD = 10240
ROWS_W = NPAD // NS
K = 128
EW = 10240
EPAD = NW * EW
NCHUNK = EW // K
KG = 80
NCG = EW // KG

f32 = jnp.float32
i32 = jnp.int32

_mesh = plsc.VectorSubcoreMesh(core_axis_name="c", subcore_axis_name="s",
                               num_cores=NC, num_subcores=NS)


@functools.partial(
    pl.kernel,
    out_type=jax.ShapeDtypeStruct((NC * NPAD, 16), f32),
    mesh=_mesh,
    compiler_params=pltpu.CompilerParams(use_tc_tiling_on_sc=False),
    scratch_types=[
        pltpu.VMEM((K,), i32),
        pltpu.VMEM((K, 16), f32),
        pltpu.VMEM((K, 16), f32),
        pltpu.VMEM_SHARED((NPAD, 16), f32),
    ],
)
def _sc_deg(dst_hbm, out_hbm, idx_d, ones_b, zero_b, deg_sh):
    cid = lax.axis_index("c")
    sid = lax.axis_index("s")
    wid = cid * NS + sid

    def fill(j, _):
        ones_b[j] = jnp.ones((16,), f32)
        zero_b[j] = jnp.zeros((16,), f32)
        return 0
    lax.fori_loop(0, K, fill, 0)

    r0 = sid * ROWS_W
    for t in range(ROWS_W // K):
        pltpu.sync_copy(zero_b, deg_sh.at[pl.ds(r0 + t * K, K)])
    plsc.subcore_barrier()

    ebase = wid * EW

    def chunk(ci, _):
        b = ebase + ci * K
        pltpu.sync_copy(dst_hbm.at[pl.ds(b, K)], idx_d)
        pltpu.sync_copy(ones_b, deg_sh.at[idx_d], add=True)
        return 0
    lax.fori_loop(0, NCHUNK, chunk, 0)

    plsc.subcore_barrier()
    o0 = cid * NPAD + r0
    for t in range(ROWS_W // K):
        pltpu.sync_copy(deg_sh.at[pl.ds(r0 + t * K, K)],
                        out_hbm.at[pl.ds(o0 + t * K, K)])


@functools.partial(
    pl.kernel,
    out_type=jax.ShapeDtypeStruct((NC * NPAD, 16), f32),
    mesh=_mesh,
    compiler_params=pltpu.CompilerParams(use_tc_tiling_on_sc=False),
    scratch_types=[
        pltpu.VMEM((K,), i32),
        pltpu.VMEM((K,), i32),
        pltpu.VMEM((K, 16), f32),
        pltpu.VMEM_SHARED((NPAD, 16), f32),
        pltpu.SemaphoreType.DMA,
    ],
)
def _sc_q(src_hbm, dst_hbm, p_hbm, out_hbm, idx_s, idx_d, buf, q_sh, sem):
    cid = lax.axis_index("c")
    sid = lax.axis_index("s")
    wid = cid * NS + sid

    def fill(j, _):
        buf[j] = jnp.zeros((16,), f32)
        return 0
    lax.fori_loop(0, K, fill, 0)

    r0 = sid * ROWS_W
    for t in range(ROWS_W // K):
        pltpu.sync_copy(buf, q_sh.at[pl.ds(r0 + t * K, K)])
    plsc.subcore_barrier()

    ebase = wid * EW

    def chunk(ci, _):
        b = ebase + ci * K
        pltpu.sync_copy(src_hbm.at[pl.ds(b, K)], idx_s)
        pltpu.sync_copy(dst_hbm.at[pl.ds(b, K)], idx_d)
        pltpu.async_copy(p_hbm.at[idx_s], buf, sem).wait()
        pltpu.sync_copy(buf, q_sh.at[idx_d], add=True)
        return 0
    lax.fori_loop(0, NCHUNK, chunk, 0)

    plsc.subcore_barrier()
    o0 = cid * NPAD + r0
    for t in range(ROWS_W // K):
        pltpu.sync_copy(q_sh.at[pl.ds(r0 + t * K, K)],
                        out_hbm.at[pl.ds(o0 + t * K, K)])


@functools.partial(
    pl.kernel,
    out_type=(jax.ShapeDtypeStruct((NC * NPAD, FEAT), f32),
              jax.ShapeDtypeStruct((NC * NPAD, 16), f32)),
    mesh=_mesh,
    compiler_params=pltpu.CompilerParams(use_tc_tiling_on_sc=False),
    scratch_types=[
        pltpu.VMEM((KG,), i32),
        pltpu.VMEM((KG,), i32),
        pltpu.VMEM((KG,), i32),
        pltpu.VMEM((KG,), i32),
        pltpu.VMEM((KG, 16), f32),
        pltpu.VMEM((KG, 16), f32),
        pltpu.VMEM((KG, 16), f32),
        pltpu.VMEM((KG, 16), f32),
        pltpu.VMEM((KG, FEAT), f32),
        pltpu.VMEM((KG, FEAT), f32),
        pltpu.VMEM((KG, 16), f32),
        pltpu.VMEM((KG, 16), f32),
        pltpu.VMEM_SHARED((NPAD, FEAT), f32),
        pltpu.VMEM_SHARED((NPAD, 16), f32),
        pltpu.SemaphoreType.DMA,
        pltpu.SemaphoreType.DMA,
    ],
)
def _sc_gat(src_hbm, dst_hbm, xw_hbm, ts_hbm, td_hbm, acc_out, den_out,
            is_a, is_c, id_a, id_c, ts_a, ts_c, td_a, td_c, rows_a, rows_c,
            ex_a, ex_c, acc_sh, den_sh, sem_a, sem_c):
    cid = lax.axis_index("c")
    sid = lax.axis_index("s")
    wid = cid * NS + sid

    def zfill(j, _):
        for h in range(FEAT // 16):
            rows_a[j, pl.ds(h * 16, 16)] = jnp.zeros((16,), f32)
        ex_a[j] = jnp.zeros((16,), f32)
        return 0
    lax.fori_loop(0, KG, zfill, 0)

    r0 = sid * ROWS_W
    for t in range(ROWS_W // KG):
        pltpu.sync_copy(rows_a, acc_sh.at[pl.ds(r0 + t * KG, KG)])
        pltpu.sync_copy(ex_a, den_sh.at[pl.ds(r0 + t * KG, KG)])
    plsc.subcore_barrier()

    cbase = wid * NCG
    hvec = [jnp.full((16,), h, i32) for h in range(H)]

    def idx_load(ci, is_s, id_s, sem_s):
        pltpu.async_copy(src_hbm.at[ci + cbase], is_s, sem_s)
        pltpu.async_copy(dst_hbm.at[ci + cbase], id_s, sem_s)

    def idx_wait(ci, is_s, id_s, sem_s):
        pltpu.make_async_copy(src_hbm.at[ci + cbase], is_s, sem_s).wait()
        pltpu.make_async_copy(dst_hbm.at[ci + cbase], id_s, sem_s).wait()

    def gat_issue(is_s, id_s, ts_s, td_s, rows_s, sem_s):
        pltpu.async_copy(ts_hbm.at[is_s], ts_s, sem_s)
        pltpu.async_copy(td_hbm.at[id_s], td_s, sem_s)
        pltpu.async_copy(xw_hbm.at[is_s], rows_s, sem_s)

    def gat_wait(is_s, id_s, ts_s, td_s, rows_s, sem_s):
        pltpu.make_async_copy(ts_hbm.at[is_s], ts_s, sem_s).wait()
        pltpu.make_async_copy(td_hbm.at[id_s], td_s, sem_s).wait()
        pltpu.make_async_copy(xw_hbm.at[is_s], rows_s, sem_s).wait()

    slot_a = (is_a, id_a, ts_a, td_a, rows_a, ex_a, sem_a)
    slot_c = (is_c, id_c, ts_c, td_c, rows_c, ex_c, sem_c)

    idx_load(0, is_a, id_a, sem_a)
    idx_wait(0, is_a, id_a, sem_a)
    gat_issue(is_a, id_a, ts_a, td_a, rows_a, sem_a)
    idx_load(1, is_c, id_c, sem_c)

    def half(ci, cur, nxt):
        is_s, id_s, ts_s, td_s, rows_s, ex_s, sem_s = cur
        is_n, id_n, ts_n, td_n, rows_n, ex_n, sem_n = nxt

        @pl.when(ci + 1 < NCG)
        def _():
            idx_wait(ci + 1, is_n, id_n, sem_n)
            gat_issue(is_n, id_n, ts_n, td_n, rows_n, sem_n)

        gat_wait(is_s, id_s, ts_s, td_s, rows_s, sem_s)

        def edge2(j2, _):
            for u in range(2):
                j = 2 * j2 + u
                a = ts_s[j] + td_s[j]
                ex = jnp.exp(jnp.maximum(a, 0.2 * a))
                ex_s[j] = ex
                for h in range(H):
                    sc16 = ex.at[hvec[h]].get(mode="promise_in_bounds")
                    rows_s[j, pl.ds(h * 16, 16)] = (
                        rows_s[j, pl.ds(h * 16, 16)] * sc16)
            return 0
        lax.fori_loop(0, KG // 2, edge2, 0)

        pltpu.sync_copy(ex_s, den_sh.at[id_s], add=True)
        pltpu.sync_copy(rows_s, acc_sh.at[id_s], add=True)

        @pl.when(ci + 2 < NCG)
        def _():
            idx_load(ci + 2, is_s, id_s, sem_s)

    def pair(g2, _):
        half(2 * g2, slot_a, slot_c)
        half(2 * g2 + 1, slot_c, slot_a)
        return 0
    lax.fori_loop(0, NCG // 2, pair, 0)

    plsc.subcore_barrier()
    o0 = cid * NPAD + r0
    for t in range(ROWS_W // K):
        pltpu.sync_copy(acc_sh.at[pl.ds(r0 + t * K, K)],
                        acc_out.at[pl.ds(o0 + t * K, K)])
        pltpu.sync_copy(den_sh.at[pl.ds(r0 + t * K, K)],
                        den_out.at[pl.ds(o0 + t * K, K)])


def _tc_prep_body(dega, sig, p_ref):
    deg = dega[0:N, 0:1] + dega[NPAD:NPAD + N, 0:1] + 1.0
    dis = lax.rsqrt(deg)
    p = dis * sig[...]
    p_ref[...] = jnp.broadcast_to(p, (N, 16))


def _tc_gcn_body(qa, dega, sig, gcnW, gcnb, gnw, gnb, gnms, x_ref):
    deg = dega[0:N, 0:1] + dega[NPAD:NPAD + N, 0:1] + 1.0
    dis = lax.rsqrt(deg)
    s = sig[...]
    q = qa[0:N, 0:1] + qa[NPAD:NPAD + N, 0:1]
    agg = dis * q + dis * dis * s
    x = jnp.maximum(agg * gcnW[...] + gcnb[...], 0.0)
    mean = jnp.mean(x, axis=0, keepdims=True)
    o = x - mean * gnms[...]
    var = jnp.mean(o * o, axis=0, keepdims=True)
    x_ref[...] = gnw[...] * o / jnp.sqrt(var + 1e-5) + gnb[...]


def _tc_proj_body(x, Wt, Asr, Adr, xw_ref, ts_ref, td_ref):
    xw = jnp.dot(x[...], Wt[...], preferred_element_type=f32)
    xw_ref[...] = xw
    ts_ref[...] = jnp.dot(xw, Asr[...], preferred_element_type=f32)
    td_ref[...] = jnp.dot(xw, Adr[...], preferred_element_type=f32)


def _tc_norm_body(x, xw, accp, denp, ts, td, bias, R16, xn_ref):
    a = ts[...] + td[...]
    exs = jnp.exp(jnp.maximum(a, 0.2 * a))
    den = denp[0:N] + denp[NPAD:NPAD + N] + exs
    inv = 1.0 / den
    acc = (accp[0:N] + accp[NPAD:NPAD + N]
           + jnp.dot(exs, R16[...], preferred_element_type=f32) * xw[...])
    g = acc * jnp.dot(inv, R16[...], preferred_element_type=f32)
    xn_ref[...] = x[...] + jnp.maximum(g + bias[...], 0.0)


def _tc_lin_body(x, Wt, b, y_ref):
    y_ref[...] = jnp.dot(x[...], Wt[...], preferred_element_type=f32) + b[...]


def _tc_prep(dega, sig):
    return pl.pallas_call(
        _tc_prep_body,
        out_shape=jax.ShapeDtypeStruct((N, 16), f32))(dega, sig)


def _tc_gcn(qa, dega, sig, gcnW, gcnb, gnw, gnb, gnms):
    return pl.pallas_call(
        _tc_gcn_body,
        out_shape=jax.ShapeDtypeStruct((N, FEAT), f32))(
            qa, dega, sig, gcnW, gcnb, gnw, gnb, gnms)


def _tc_proj(x, Wt, Asr, Adr):
    shp = (jax.ShapeDtypeStruct((N, FEAT), f32),
           jax.ShapeDtypeStruct((N, 16), f32),
           jax.ShapeDtypeStruct((N, 16), f32))
    return pl.pallas_call(_tc_proj_body, out_shape=shp)(x, Wt, Asr, Adr)


def _tc_norm(x, xw, accp, denp, ts, td, bias, R16):
    return pl.pallas_call(
        _tc_norm_body,
        out_shape=jax.ShapeDtypeStruct((N, FEAT), f32))(
            x, xw, accp, denp, ts, td, bias, R16)


def _tc_lin(x, Wt, b):
    return pl.pallas_call(
        _tc_lin_body,
        out_shape=jax.ShapeDtypeStruct((N, OUT), f32))(x, Wt, b)


def kernel(signals, edge_index, gcn_W, gcn_b, gn_w, gn_b, gn_ms,
           gat0_W, gat0_as, gat0_ad, gat0_b,
           gat1_W, gat1_as, gat1_ad,
           gat2_W, gat2_as, gat2_ad, lin_W, lin_b):
    src = edge_index[0].astype(i32)
    dst = edge_index[1].astype(i32)
    padw = EW - E // NW
    src_w = src.reshape(NW, E // NW)
    dst_w = dst.reshape(NW, E // NW)
    pad_dst = jnp.broadcast_to(N + jnp.arange(padw, dtype=i32), (NW, padw))
    srcp = jnp.concatenate([src_w, jnp.zeros((NW, padw), i32)], 1).reshape(-1)
    dstp = jnp.concatenate([dst_w, pad_dst], 1).reshape(-1)
    src2d = srcp.reshape(EPAD // KG, KG)
    dst2d = dstp.reshape(EPAD // KG, KG)

    eye = jnp.eye(H, dtype=f32)

    def amat(a):
        A1 = (eye[:, None, :] * a[:, :, None]).reshape(FEAT, H)
        return jnp.concatenate([A1, A1], axis=1)

    As0, Ad0 = amat(gat0_as), amat(gat0_ad)
    As1, Ad1 = amat(gat1_as), amat(gat1_ad)
    As2, Ad2 = amat(gat2_as), amat(gat2_ad)
    R16 = jnp.concatenate([jnp.repeat(eye, C, axis=1),
                           jnp.zeros((H, FEAT), f32)], axis=0)

    dega = _sc_deg(dstp)
    p_tab = _tc_prep(dega, signals)
    qa = _sc_q(srcp, dstp, p_tab)
    x0 = _tc_gcn(qa, dega, signals, gcn_W, gcn_b.reshape(1, FEAT),
                 gn_w.reshape(1, FEAT), gn_b.reshape(1, FEAT),
                 gn_ms.reshape(1, FEAT))
    zbias = jnp.zeros((1, FEAT), f32)
    x = x0
    for Wt, Asr, Adr, bias in (
            (gat0_W.T, As0, Ad0, gat0_b.reshape(1, FEAT)),
            (gat1_W.T, As1, Ad1, zbias),
            (gat2_W.T, As2, Ad2, zbias)):
        xw, ts, td = _tc_proj(x, Wt, Asr, Adr)
        accp, denp = _sc_gat(src2d, dst2d, xw, ts, td)
        x = _tc_norm(x, xw, accp, denp, ts, td, bias, R16)
    return _tc_lin(x, lin_W.T, lin_b.reshape(1, OUT))


# speedup vs baseline: 1.0034x; 1.0034x over previous
import functools

import jax
import jax.numpy as jnp
from jax import lax
from jax.experimental import pallas as pl
from jax.experimental.pallas import tpu as pltpu
from jax.experimental.pallas import tpu_sc as plsc

N = 10000
E = 320000
FEAT = 128
H = 8
C = 16
OUT = 64

NC = 2
NS = 16
NW = NC * NS
NPAD = 10240
ROWS_W = NPAD // NS
K = 128
EW = 10240
EPAD = NW * EW
NCHUNK = EW // K
KG = 80
NCG = EW // KG

f32 = jnp.float32
i32 = jnp.int32

_mesh = plsc.VectorSubcoreMesh(core_axis_name="c", subcore_axis_name="s",
                               num_cores=NC, num_subcores=NS)


@functools.partial(
    pl.kernel,
    out_type=jax.ShapeDtypeStruct((NC * NPAD, 16), f32),
    mesh=_mesh,
    compiler_params=pltpu.CompilerParams(use_tc_tiling_on_sc=False),
    scratch_types=[
        pltpu.VMEM((K,), i32),
        pltpu.VMEM((K, 16), f32),
        pltpu.VMEM((K, 16), f32),
        pltpu.VMEM_SHARED((NPAD, 16), f32),
    ],
)
def _sc_deg(dst_hbm, out_hbm, idx_d, ones_b, zero_b, deg_sh):
    cid = lax.axis_index("c")
    sid = lax.axis_index("s")
    wid = cid * NS + sid

    def fill(j, _):
        ones_b[j] = jnp.ones((16,), f32)
        zero_b[j] = jnp.zeros((16,), f32)
        return 0
    lax.fori_loop(0, K, fill, 0)

    r0 = sid * ROWS_W
    for t in range(ROWS_W // K):
        pltpu.sync_copy(zero_b, deg_sh.at[pl.ds(r0 + t * K, K)])
    plsc.subcore_barrier()

    ebase = wid * EW

    def chunk(ci, _):
        b = ebase + ci * K
        pltpu.sync_copy(dst_hbm.at[pl.ds(b, K)], idx_d)
        pltpu.sync_copy(ones_b, deg_sh.at[idx_d], add=True)
        return 0
    lax.fori_loop(0, NCHUNK, chunk, 0)

    plsc.subcore_barrier()
    o0 = cid * NPAD + r0
    for t in range(ROWS_W // K):
        pltpu.sync_copy(deg_sh.at[pl.ds(r0 + t * K, K)],
                        out_hbm.at[pl.ds(o0 + t * K, K)])


@functools.partial(
    pl.kernel,
    out_type=jax.ShapeDtypeStruct((NC * NPAD, 16), f32),
    mesh=_mesh,
    compiler_params=pltpu.CompilerParams(use_tc_tiling_on_sc=False),
    scratch_types=[
        pltpu.VMEM((K,), i32),
        pltpu.VMEM((K,), i32),
        pltpu.VMEM((K, 16), f32),
        pltpu.VMEM_SHARED((NPAD, 16), f32),
        pltpu.SemaphoreType.DMA,
    ],
)
def _sc_q(src_hbm, dst_hbm, p_hbm, out_hbm, idx_s, idx_d, buf, q_sh, sem):
    cid = lax.axis_index("c")
    sid = lax.axis_index("s")
    wid = cid * NS + sid

    def fill(j, _):
        buf[j] = jnp.zeros((16,), f32)
        return 0
    lax.fori_loop(0, K, fill, 0)

    r0 = sid * ROWS_W
    for t in range(ROWS_W // K):
        pltpu.sync_copy(buf, q_sh.at[pl.ds(r0 + t * K, K)])
    plsc.subcore_barrier()

    ebase = wid * EW

    def chunk(ci, _):
        b = ebase + ci * K
        pltpu.sync_copy(src_hbm.at[pl.ds(b, K)], idx_s)
        pltpu.sync_copy(dst_hbm.at[pl.ds(b, K)], idx_d)
        pltpu.async_copy(p_hbm.at[idx_s], buf, sem).wait()
        pltpu.sync_copy(buf, q_sh.at[idx_d], add=True)
        return 0
    lax.fori_loop(0, NCHUNK, chunk, 0)

    plsc.subcore_barrier()
    o0 = cid * NPAD + r0
    for t in range(ROWS_W // K):
        pltpu.sync_copy(q_sh.at[pl.ds(r0 + t * K, K)],
                        out_hbm.at[pl.ds(o0 + t * K, K)])


@functools.partial(
    pl.kernel,
    out_type=(jax.ShapeDtypeStruct((NC * NPAD, FEAT), f32),
              jax.ShapeDtypeStruct((NC * NPAD, 16), f32)),
    mesh=_mesh,
    compiler_params=pltpu.CompilerParams(use_tc_tiling_on_sc=False),
    scratch_types=[
        pltpu.VMEM((KG,), i32),
        pltpu.VMEM((KG,), i32),
        pltpu.VMEM((KG,), i32),
        pltpu.VMEM((KG,), i32),
        pltpu.VMEM((KG, 16), f32),
        pltpu.VMEM((KG, 16), f32),
        pltpu.VMEM((KG, 16), f32),
        pltpu.VMEM((KG, 16), f32),
        pltpu.VMEM((KG, FEAT), f32),
        pltpu.VMEM((KG, FEAT), f32),
        pltpu.VMEM((KG, 16), f32),
        pltpu.VMEM((KG, 16), f32),
        pltpu.VMEM_SHARED((NPAD, FEAT), f32),
        pltpu.VMEM_SHARED((NPAD, 16), f32),
        pltpu.SemaphoreType.DMA,
        pltpu.SemaphoreType.DMA,
    ],
)
def _sc_gat(src_hbm, dst_hbm, xw_hbm, ts_hbm, td_hbm, acc_out, den_out,
            is_a, is_c, id_a, id_c, ts_a, ts_c, td_a, td_c, rows_a, rows_c,
            ex_a, ex_c, acc_sh, den_sh, sem_a, sem_c):
    cid = lax.axis_index("c")
    sid = lax.axis_index("s")
    wid = cid * NS + sid

    def zfill(j, _):
        for h in range(FEAT // 16):
            rows_a[j, pl.ds(h * 16, 16)] = jnp.zeros((16,), f32)
        ex_a[j] = jnp.zeros((16,), f32)
        return 0
    lax.fori_loop(0, KG, zfill, 0)

    r0 = sid * ROWS_W
    for t in range(ROWS_W // KG):
        pltpu.sync_copy(rows_a, acc_sh.at[pl.ds(r0 + t * KG, KG)])
        pltpu.sync_copy(ex_a, den_sh.at[pl.ds(r0 + t * KG, KG)])
    plsc.subcore_barrier()

    cbase = wid * NCG
    hvec = [jnp.full((16,), h, i32) for h in range(H)]

    def idx_load(ci, is_s, id_s, sem_s):
        pltpu.async_copy(src_hbm.at[ci + cbase], is_s, sem_s)
        pltpu.async_copy(dst_hbm.at[ci + cbase], id_s, sem_s)

    def idx_wait(ci, is_s, id_s, sem_s):
        pltpu.make_async_copy(src_hbm.at[ci + cbase], is_s, sem_s).wait()
        pltpu.make_async_copy(dst_hbm.at[ci + cbase], id_s, sem_s).wait()

    def gat_issue(is_s, id_s, ts_s, td_s, rows_s, sem_s):
        pltpu.async_copy(ts_hbm.at[is_s], ts_s, sem_s)
        pltpu.async_copy(td_hbm.at[id_s], td_s, sem_s)
        pltpu.async_copy(xw_hbm.at[is_s], rows_s, sem_s)

    def gat_wait(is_s, id_s, ts_s, td_s, rows_s, sem_s):
        pltpu.make_async_copy(ts_hbm.at[is_s], ts_s, sem_s).wait()
        pltpu.make_async_copy(td_hbm.at[id_s], td_s, sem_s).wait()
        pltpu.make_async_copy(xw_hbm.at[is_s], rows_s, sem_s).wait()

    slot_a = (is_a, id_a, ts_a, td_a, rows_a, ex_a, sem_a)
    slot_c = (is_c, id_c, ts_c, td_c, rows_c, ex_c, sem_c)

    idx_load(0, is_a, id_a, sem_a)
    idx_wait(0, is_a, id_a, sem_a)
    gat_issue(is_a, id_a, ts_a, td_a, rows_a, sem_a)
    idx_load(1, is_c, id_c, sem_c)

    def half(ci, cur, nxt):
        is_s, id_s, ts_s, td_s, rows_s, ex_s, sem_s = cur
        is_n, id_n, ts_n, td_n, rows_n, ex_n, sem_n = nxt

        @pl.when(ci + 1 < NCG)
        def _():
            idx_wait(ci + 1, is_n, id_n, sem_n)
            gat_issue(is_n, id_n, ts_n, td_n, rows_n, sem_n)

        gat_wait(is_s, id_s, ts_s, td_s, rows_s, sem_s)

        def edge(j, _):
            a = ts_s[j] + td_s[j]
            ex = jnp.exp(jnp.maximum(a, 0.2 * a))
            ex_s[j] = ex
            for h in range(H):
                sc16 = ex.at[hvec[h]].get(mode="promise_in_bounds")
                rows_s[j, pl.ds(h * 16, 16)] = (
                    rows_s[j, pl.ds(h * 16, 16)] * sc16)
            return 0
        lax.fori_loop(0, KG, edge, 0)

        pltpu.sync_copy(ex_s, den_sh.at[id_s], add=True)
        pltpu.sync_copy(rows_s, acc_sh.at[id_s], add=True)

        @pl.when(ci + 2 < NCG)
        def _():
            idx_load(ci + 2, is_s, id_s, sem_s)

    def pair(g2, _):
        half(2 * g2, slot_a, slot_c)
        half(2 * g2 + 1, slot_c, slot_a)
        return 0
    lax.fori_loop(0, NCG // 2, pair, 0)

    plsc.subcore_barrier()
    o0 = cid * NPAD + r0
    for t in range(ROWS_W // K):
        pltpu.sync_copy(acc_sh.at[pl.ds(r0 + t * K, K)],
                        acc_out.at[pl.ds(o0 + t * K, K)])
        pltpu.sync_copy(den_sh.at[pl.ds(r0 + t * K, K)],
                        den_out.at[pl.ds(o0 + t * K, K)])


def _tc_prep_body(dega, sig, p_ref):
    deg = dega[0:N, 0:1] + dega[NPAD:NPAD + N, 0:1] + 1.0
    dis = lax.rsqrt(deg)
    p = dis * sig[...]
    p_ref[...] = jnp.broadcast_to(p, (N, 16))


def _tc_gcn_body(qa, dega, sig, gcnW, gcnb, gnw, gnb, gnms, x_ref):
    deg = dega[0:N, 0:1] + dega[NPAD:NPAD + N, 0:1] + 1.0
    dis = lax.rsqrt(deg)
    s = sig[...]
    q = qa[0:N, 0:1] + qa[NPAD:NPAD + N, 0:1]
    agg = dis * q + dis * dis * s
    x = jnp.maximum(agg * gcnW[...] + gcnb[...], 0.0)
    mean = jnp.mean(x, axis=0, keepdims=True)
    o = x - mean * gnms[...]
    var = jnp.mean(o * o, axis=0, keepdims=True)
    x_ref[...] = gnw[...] * o / jnp.sqrt(var + 1e-5) + gnb[...]


def _tc_proj_body(x, Wt, Asr, Adr, xw_ref, ts_ref, td_ref):
    xw = jnp.dot(x[...], Wt[...], preferred_element_type=f32)
    xw_ref[...] = xw
    ts_ref[...] = jnp.dot(xw, Asr[...], preferred_element_type=f32)
    td_ref[...] = jnp.dot(xw, Adr[...], preferred_element_type=f32)


def _tc_norm_body(x, xw, accp, denp, ts, td, bias, R16, xn_ref):
    a = ts[...] + td[...]
    exs = jnp.exp(jnp.maximum(a, 0.2 * a))
    den = denp[0:N] + denp[NPAD:NPAD + N] + exs
    inv = 1.0 / den
    acc = (accp[0:N] + accp[NPAD:NPAD + N]
           + jnp.dot(exs, R16[...], preferred_element_type=f32) * xw[...])
    g = acc * jnp.dot(inv, R16[...], preferred_element_type=f32)
    xn_ref[...] = x[...] + jnp.maximum(g + bias[...], 0.0)


def _tc_lin_body(x, Wt, b, y_ref):
    y_ref[...] = jnp.dot(x[...], Wt[...], preferred_element_type=f32) + b[...]


def _tc_prep(dega, sig):
    return pl.pallas_call(
        _tc_prep_body,
        out_shape=jax.ShapeDtypeStruct((N, 16), f32))(dega, sig)


def _tc_gcn(qa, dega, sig, gcnW, gcnb, gnw, gnb, gnms):
    return pl.pallas_call(
        _tc_gcn_body,
        out_shape=jax.ShapeDtypeStruct((N, FEAT), f32))(
            qa, dega, sig, gcnW, gcnb, gnw, gnb, gnms)


def _tc_proj(x, Wt, Asr, Adr):
    shp = (jax.ShapeDtypeStruct((N, FEAT), f32),
           jax.ShapeDtypeStruct((N, 16), f32),
           jax.ShapeDtypeStruct((N, 16), f32))
    return pl.pallas_call(_tc_proj_body, out_shape=shp)(x, Wt, Asr, Adr)


def _tc_norm(x, xw, accp, denp, ts, td, bias, R16):
    return pl.pallas_call(
        _tc_norm_body,
        out_shape=jax.ShapeDtypeStruct((N, FEAT), f32))(
            x, xw, accp, denp, ts, td, bias, R16)


def _tc_lin(x, Wt, b):
    return pl.pallas_call(
        _tc_lin_body,
        out_shape=jax.ShapeDtypeStruct((N, OUT), f32))(x, Wt, b)


def kernel(signals, edge_index, gcn_W, gcn_b, gn_w, gn_b, gn_ms,
           gat0_W, gat0_as, gat0_ad, gat0_b,
           gat1_W, gat1_as, gat1_ad,
           gat2_W, gat2_as, gat2_ad, lin_W, lin_b):
    src = edge_index[0].astype(i32)
    dst = edge_index[1].astype(i32)
    padw = EW - E // NW
    src_w = src.reshape(NW, E // NW)
    dst_w = dst.reshape(NW, E // NW)
    pad_dst = jnp.broadcast_to(N + jnp.arange(padw, dtype=i32), (NW, padw))
    srcp = jnp.concatenate([src_w, jnp.zeros((NW, padw), i32)], 1).reshape(-1)
    dstp = jnp.concatenate([dst_w, pad_dst], 1).reshape(-1)
    src2d = srcp.reshape(EPAD // KG, KG)
    dst2d = dstp.reshape(EPAD // KG, KG)

    eye = jnp.eye(H, dtype=f32)

    def amat(a):
        A1 = (eye[:, None, :] * a[:, :, None]).reshape(FEAT, H)
        return jnp.concatenate([A1, A1], axis=1)

    As0, Ad0 = amat(gat0_as), amat(gat0_ad)
    As1, Ad1 = amat(gat1_as), amat(gat1_ad)
    As2, Ad2 = amat(gat2_as), amat(gat2_ad)
    R16 = jnp.concatenate([jnp.repeat(eye, C, axis=1),
                           jnp.zeros((H, FEAT), f32)], axis=0)

    dega = _sc_deg(dstp)
    p_tab = _tc_prep(dega, signals)
    qa = _sc_q(srcp, dstp, p_tab)
    x0 = _tc_gcn(qa, dega, signals, gcn_W, gcn_b.reshape(1, FEAT),
                 gn_w.reshape(1, FEAT), gn_b.reshape(1, FEAT),
                 gn_ms.reshape(1, FEAT))
    zbias = jnp.zeros((1, FEAT), f32)
    x = x0
    for Wt, Asr, Adr, bias in (
            (gat0_W.T, As0, Ad0, gat0_b.reshape(1, FEAT)),
            (gat1_W.T, As1, Ad1, zbias),
            (gat2_W.T, As2, Ad2, zbias)):
        xw, ts, td = _tc_proj(x, Wt, Asr, Adr)
        accp, denp = _sc_gat(src2d, dst2d, xw, ts, td)
        x = _tc_norm(x, xw, accp, denp, ts, td, bias, R16)
    return _tc_lin(x, lin_W.T, lin_b.reshape(1, OUT))


# speedup vs baseline: 1.1589x; 1.1550x over previous
import functools

import jax
import jax.numpy as jnp
from jax import lax
from jax.experimental import pallas as pl
from jax.experimental.pallas import tpu as pltpu
from jax.experimental.pallas import tpu_sc as plsc

N = 10000
E = 320000
FEAT = 128
H = 8
C = 16
OUT = 64

NC = 2
NS = 16
NW = NC * NS
NPAD = 10240
ROWS_W = NPAD // NS
K = 128
EW = 10240
EPAD = NW * EW
NCHUNK = EW // K
KG = 80
NCG = EW // KG

f32 = jnp.float32
i32 = jnp.int32

_mesh = plsc.VectorSubcoreMesh(core_axis_name="c", subcore_axis_name="s",
                               num_cores=NC, num_subcores=NS)


@functools.partial(
    pl.kernel,
    out_type=jax.ShapeDtypeStruct((NC * NPAD, 16), f32),
    mesh=_mesh,
    compiler_params=pltpu.CompilerParams(use_tc_tiling_on_sc=False),
    scratch_types=[
        pltpu.VMEM((K,), i32),
        pltpu.VMEM((K, 16), f32),
        pltpu.VMEM((K, 16), f32),
        pltpu.VMEM_SHARED((NPAD, 16), f32),
    ],
)
def _sc_deg(dst_hbm, out_hbm, idx_d, ones_b, zero_b, deg_sh):
    cid = lax.axis_index("c")
    sid = lax.axis_index("s")
    wid = cid * NS + sid

    def fill(j, _):
        ones_b[j] = jnp.ones((16,), f32)
        zero_b[j] = jnp.zeros((16,), f32)
        return 0
    lax.fori_loop(0, K, fill, 0)

    r0 = sid * ROWS_W
    for t in range(ROWS_W // K):
        pltpu.sync_copy(zero_b, deg_sh.at[pl.ds(r0 + t * K, K)])
    plsc.subcore_barrier()

    ebase = wid * EW

    def chunk(ci, _):
        b = ebase + ci * K
        pltpu.sync_copy(dst_hbm.at[pl.ds(b, K)], idx_d)
        pltpu.sync_copy(ones_b, deg_sh.at[idx_d], add=True)
        return 0
    lax.fori_loop(0, NCHUNK, chunk, 0)

    plsc.subcore_barrier()
    o0 = cid * NPAD + r0
    for t in range(ROWS_W // K):
        pltpu.sync_copy(deg_sh.at[pl.ds(r0 + t * K, K)],
                        out_hbm.at[pl.ds(o0 + t * K, K)])


@functools.partial(
    pl.kernel,
    out_type=jax.ShapeDtypeStruct((NC * NPAD, 16), f32),
    mesh=_mesh,
    compiler_params=pltpu.CompilerParams(use_tc_tiling_on_sc=False),
    scratch_types=[
        pltpu.VMEM((K,), i32),
        pltpu.VMEM((K,), i32),
        pltpu.VMEM((K, 16), f32),
        pltpu.VMEM_SHARED((NPAD, 16), f32),
        pltpu.SemaphoreType.DMA,
    ],
)
def _sc_q(src_hbm, dst_hbm, p_hbm, out_hbm, idx_s, idx_d, buf, q_sh, sem):
    cid = lax.axis_index("c")
    sid = lax.axis_index("s")
    wid = cid * NS + sid

    def fill(j, _):
        buf[j] = jnp.zeros((16,), f32)
        return 0
    lax.fori_loop(0, K, fill, 0)

    r0 = sid * ROWS_W
    for t in range(ROWS_W // K):
        pltpu.sync_copy(buf, q_sh.at[pl.ds(r0 + t * K, K)])
    plsc.subcore_barrier()

    ebase = wid * EW

    def chunk(ci, _):
        b = ebase + ci * K
        pltpu.sync_copy(src_hbm.at[pl.ds(b, K)], idx_s)
        pltpu.sync_copy(dst_hbm.at[pl.ds(b, K)], idx_d)
        pltpu.async_copy(p_hbm.at[idx_s], buf, sem).wait()
        pltpu.sync_copy(buf, q_sh.at[idx_d], add=True)
        return 0
    lax.fori_loop(0, NCHUNK, chunk, 0)

    plsc.subcore_barrier()
    o0 = cid * NPAD + r0
    for t in range(ROWS_W // K):
        pltpu.sync_copy(q_sh.at[pl.ds(r0 + t * K, K)],
                        out_hbm.at[pl.ds(o0 + t * K, K)])


@functools.partial(
    pl.kernel,
    out_type=(jax.ShapeDtypeStruct((NC * NPAD, FEAT), f32),
              jax.ShapeDtypeStruct((NC * NPAD, 16), f32)),
    mesh=_mesh,
    compiler_params=pltpu.CompilerParams(use_tc_tiling_on_sc=False),
    scratch_types=[
        pltpu.VMEM((KG,), i32),
        pltpu.VMEM((KG,), i32),
        pltpu.VMEM((KG,), i32),
        pltpu.VMEM((KG,), i32),
        pltpu.VMEM((KG, 16), f32),
        pltpu.VMEM((KG, 16), f32),
        pltpu.VMEM((KG, 16), f32),
        pltpu.VMEM((KG, 16), f32),
        pltpu.VMEM((KG, FEAT), f32),
        pltpu.VMEM((KG, FEAT), f32),
        pltpu.VMEM((KG, 16), f32),
        pltpu.VMEM((KG, 16), f32),
        pltpu.VMEM_SHARED((NPAD, FEAT), f32),
        pltpu.VMEM_SHARED((NPAD, 16), f32),
        pltpu.SemaphoreType.DMA,
        pltpu.SemaphoreType.DMA,
    ],
)
def _sc_gat(src_hbm, dst_hbm, xw_hbm, ts_hbm, td_hbm, acc_out, den_out,
            is_a, is_c, id_a, id_c, ts_a, ts_c, td_a, td_c, rows_a, rows_c,
            ex_a, ex_c, acc_sh, den_sh, sem_a, sem_c):
    cid = lax.axis_index("c")
    sid = lax.axis_index("s")
    wid = cid * NS + sid

    def zfill(j, _):
        for h in range(FEAT // 16):
            rows_a[j, pl.ds(h * 16, 16)] = jnp.zeros((16,), f32)
        ex_a[j] = jnp.zeros((16,), f32)
        return 0
    lax.fori_loop(0, KG, zfill, 0)

    r0 = sid * ROWS_W
    for t in range(ROWS_W // KG):
        pltpu.sync_copy(rows_a, acc_sh.at[pl.ds(r0 + t * KG, KG)])
        pltpu.sync_copy(ex_a, den_sh.at[pl.ds(r0 + t * KG, KG)])
    plsc.subcore_barrier()

    cbase = wid * NCG
    hvec = [jnp.full((16,), h, i32) for h in range(H)]

    def idx_load(ci, is_s, id_s, sem_s):
        pltpu.async_copy(src_hbm.at[ci + cbase], is_s, sem_s)
        pltpu.async_copy(dst_hbm.at[ci + cbase], id_s, sem_s)

    def idx_wait(ci, is_s, id_s, sem_s):
        pltpu.make_async_copy(src_hbm.at[ci + cbase], is_s, sem_s).wait()
        pltpu.make_async_copy(dst_hbm.at[ci + cbase], id_s, sem_s).wait()

    def gat_issue(is_s, id_s, ts_s, td_s, rows_s, sem_s):
        pltpu.async_copy(ts_hbm.at[is_s], ts_s, sem_s)
        pltpu.async_copy(td_hbm.at[id_s], td_s, sem_s)
        pltpu.async_copy(xw_hbm.at[is_s], rows_s, sem_s)

    def gat_wait(is_s, id_s, ts_s, td_s, rows_s, sem_s):
        pltpu.make_async_copy(ts_hbm.at[is_s], ts_s, sem_s).wait()
        pltpu.make_async_copy(td_hbm.at[id_s], td_s, sem_s).wait()
        pltpu.make_async_copy(xw_hbm.at[is_s], rows_s, sem_s).wait()

    slot_a = (is_a, id_a, ts_a, td_a, rows_a, ex_a, sem_a)
    slot_c = (is_c, id_c, ts_c, td_c, rows_c, ex_c, sem_c)

    idx_load(0, is_a, id_a, sem_a)
    idx_wait(0, is_a, id_a, sem_a)
    gat_issue(is_a, id_a, ts_a, td_a, rows_a, sem_a)
    idx_load(1, is_c, id_c, sem_c)

    def half(ci, cur, nxt):
        is_s, id_s, ts_s, td_s, rows_s, ex_s, sem_s = cur
        is_n, id_n, ts_n, td_n, rows_n, ex_n, sem_n = nxt

        @pl.when(ci + 1 < NCG)
        def _():
            idx_wait(ci + 1, is_n, id_n, sem_n)
            gat_issue(is_n, id_n, ts_n, td_n, rows_n, sem_n)

        gat_wait(is_s, id_s, ts_s, td_s, rows_s, sem_s)

        def edge(j, _):
            a = ts_s[j] + td_s[j]
            ex = jnp.exp(jnp.maximum(a, 0.2 * a))
            ex_s[j] = ex
            for h in range(H):
                sc16 = ex.at[hvec[h]].get(mode="promise_in_bounds")
                rows_s[j, pl.ds(h * 16, 16)] = (
                    rows_s[j, pl.ds(h * 16, 16)] * sc16)
            return 0
        lax.fori_loop(0, KG, edge, 0)

        pltpu.sync_copy(ex_s, den_sh.at[id_s], add=True)
        pltpu.sync_copy(rows_s, acc_sh.at[id_s], add=True)

        @pl.when(ci + 2 < NCG)
        def _():
            idx_load(ci + 2, is_s, id_s, sem_s)

    def pair(g2, _):
        half(2 * g2, slot_a, slot_c)
        half(2 * g2 + 1, slot_c, slot_a)
        return 0
    lax.fori_loop(0, NCG // 2, pair, 0)

    plsc.subcore_barrier()
    o0 = cid * NPAD + r0
    for t in range(ROWS_W // K):
        pltpu.sync_copy(acc_sh.at[pl.ds(r0 + t * K, K)],
                        acc_out.at[pl.ds(o0 + t * K, K)])
        pltpu.sync_copy(den_sh.at[pl.ds(r0 + t * K, K)],
                        den_out.at[pl.ds(o0 + t * K, K)])


def _tc_prep_body(dega, sig, p_ref):
    deg = dega[0:N, 0:1] + dega[NPAD:NPAD + N, 0:1] + 1.0
    dis = lax.rsqrt(deg)
    p = dis * sig[...]
    p_ref[...] = jnp.broadcast_to(p, (N, 16))


def _tc_gcn_body(qa, dega, sig, gcnW, gcnb, gnw, gnb, gnms, x_ref):
    deg = dega[0:N, 0:1] + dega[NPAD:NPAD + N, 0:1] + 1.0
    dis = lax.rsqrt(deg)
    s = sig[...]
    q = qa[0:N, 0:1] + qa[NPAD:NPAD + N, 0:1]
    agg = dis * q + dis * dis * s
    x = jnp.maximum(agg * gcnW[...] + gcnb[...], 0.0)
    mean = jnp.mean(x, axis=0, keepdims=True)
    o = x - mean * gnms[...]
    var = jnp.mean(o * o, axis=0, keepdims=True)
    x_ref[...] = gnw[...] * o / jnp.sqrt(var + 1e-5) + gnb[...]


def _tc_proj_body(x, Wt, Asr, Adr, xw_ref, ts_ref, td_ref):
    xw = jnp.dot(x[...], Wt[...], preferred_element_type=f32)
    xw_ref[...] = xw
    ts_ref[...] = jnp.dot(xw, Asr[...], preferred_element_type=f32)
    td_ref[...] = jnp.dot(xw, Adr[...], preferred_element_type=f32)


def _tc_norm_body(x, xw, accp, denp, ts, td, bias, R16, xn_ref):
    a = ts[...] + td[...]
    exs = jnp.exp(jnp.maximum(a, 0.2 * a))
    den = denp[0:N] + denp[NPAD:NPAD + N] + exs
    inv = 1.0 / den
    acc = (accp[0:N] + accp[NPAD:NPAD + N]
           + jnp.dot(exs, R16[...], preferred_element_type=f32) * xw[...])
    g = acc * jnp.dot(inv, R16[...], preferred_element_type=f32)
    xn_ref[...] = x[...] + jnp.maximum(g + bias[...], 0.0)


def _tc_lin_body(x, Wt, b, y_ref):
    y_ref[...] = jnp.dot(x[...], Wt[...], preferred_element_type=f32) + b[...]


def _tc_prep(dega, sig):
    return pl.pallas_call(
        _tc_prep_body,
        out_shape=jax.ShapeDtypeStruct((N, 16), f32))(dega, sig)


def _tc_gcn(qa, dega, sig, gcnW, gcnb, gnw, gnb, gnms):
    return pl.pallas_call(
        _tc_gcn_body,
        out_shape=jax.ShapeDtypeStruct((N, FEAT), f32))(
            qa, dega, sig, gcnW, gcnb, gnw, gnb, gnms)


def _tc_proj(x, Wt, Asr, Adr):
    shp = (jax.ShapeDtypeStruct((N, FEAT), f32),
           jax.ShapeDtypeStruct((N, 16), f32),
           jax.ShapeDtypeStruct((N, 16), f32))
    return pl.pallas_call(_tc_proj_body, out_shape=shp)(x, Wt, Asr, Adr)


def _tc_norm(x, xw, accp, denp, ts, td, bias, R16):
    return pl.pallas_call(
        _tc_norm_body,
        out_shape=jax.ShapeDtypeStruct((N, FEAT), f32))(
            x, xw, accp, denp, ts, td, bias, R16)


def _tc_lin(x, Wt, b):
    return pl.pallas_call(
        _tc_lin_body,
        out_shape=jax.ShapeDtypeStruct((N, OUT), f32))(x, Wt, b)


def kernel(signals, edge_index, gcn_W, gcn_b, gn_w, gn_b, gn_ms,
           gat0_W, gat0_as, gat0_ad, gat0_b,
           gat1_W, gat1_as, gat1_ad,
           gat2_W, gat2_as, gat2_ad, lin_W, lin_b):
    src = edge_index[0].astype(i32)
    dst = edge_index[1].astype(i32)
    pad = EPAD - E
    srcp = jnp.concatenate([src, jnp.zeros((pad,), i32)])
    dstp = jnp.concatenate([dst, jnp.full((pad,), NPAD - 1, i32)])
    src2d = srcp.reshape(EPAD // KG, KG)
    dst2d = dstp.reshape(EPAD // KG, KG)

    eye = jnp.eye(H, dtype=f32)

    def amat(a):
        A1 = (eye[:, None, :] * a[:, :, None]).reshape(FEAT, H)
        return jnp.concatenate([A1, A1], axis=1)

    As0, Ad0 = amat(gat0_as), amat(gat0_ad)
    As1, Ad1 = amat(gat1_as), amat(gat1_ad)
    As2, Ad2 = amat(gat2_as), amat(gat2_ad)
    R16 = jnp.concatenate([jnp.repeat(eye, C, axis=1),
                           jnp.zeros((H, FEAT), f32)], axis=0)

    dega = _sc_deg(dstp)
    p_tab = _tc_prep(dega, signals)
    qa = _sc_q(srcp, dstp, p_tab)
    x0 = _tc_gcn(qa, dega, signals, gcn_W, gcn_b.reshape(1, FEAT),
                 gn_w.reshape(1, FEAT), gn_b.reshape(1, FEAT),
                 gn_ms.reshape(1, FEAT))
    zbias = jnp.zeros((1, FEAT), f32)
    x = x0
    for Wt, Asr, Adr, bias in (
            (gat0_W.T, As0, Ad0, gat0_b.reshape(1, FEAT)),
            (gat1_W.T, As1, Ad1, zbias),
            (gat2_W.T, As2, Ad2, zbias)):
        xw, ts, td = _tc_proj(x, Wt, Asr, Adr)
        accp, denp = _sc_gat(src2d, dst2d, xw, ts, td)
        x = _tc_norm(x, xw, accp, denp, ts, td, bias, R16)
    return _tc_lin(x, lin_W.T, lin_b.reshape(1, OUT))
